# baseline (speedup 1.0000x reference)
import functools

import jax
import jax.numpy as jnp
from jax import lax
from jax.experimental import pallas as pl
from jax.experimental.pallas import tpu as pltpu
from jax.experimental.pallas import tpu_sc as plsc

EMB = 300
EMB_PAD = 304
CHUNK = 128
NBUF = 2
PAD_BLOCK = 2048


PAD_ROWS = 125
LANES = 16
PIECES_IN = EMB // LANES


def _pad_table_sc(table):
    v = table.shape[0]
    info = plsc.get_sparse_core_info()
    nc, ns = info.num_cores, info.num_subcores
    nw = nc * ns
    per_w = v // nw
    n_tail = v - per_w * nw
    n_blk = per_w // PAD_ROWS
    assert per_w % PAD_ROWS == 0

    mesh = plsc.VectorSubcoreMesh(core_axis_name="c", subcore_axis_name="s")

    @functools.partial(
        pl.kernel,
        mesh=mesh,
        compiler_params=pltpu.CompilerParams(
            use_tc_tiling_on_sc=False, needs_layout_passes=False
        ),
        out_type=jax.ShapeDtypeStruct((v, EMB_PAD), jnp.float32),
        scratch_types=[
            pltpu.VMEM((PAD_ROWS, EMB), jnp.float32),
            pltpu.VMEM((PAD_ROWS, EMB_PAD), jnp.float32),
        ],
    )
    def k(table_hbm, out_hbm, st_in, st_out):
        wid = lax.axis_index("s") * nc + lax.axis_index("c")
        iota = lax.iota(jnp.int32, LANES)
        tail_mask = iota < (EMB - PIECES_IN * LANES)

        def pad_rows(row0, n_rows):
            pltpu.sync_copy(table_hbm.at[pl.ds(row0, n_rows)],
                            st_in.at[pl.ds(0, n_rows)])

            def row_body(r, carry):
                vals = [
                    st_in[r, pl.ds(p * LANES, LANES)] for p in range(PIECES_IN)
                ]
                tail = plsc.load_gather(
                    st_in,
                    [jnp.full((LANES,), r, jnp.int32),
                     PIECES_IN * LANES + iota],
                    mask=tail_mask,
                )
                for p in range(PIECES_IN):
                    st_out[r, pl.ds(p * LANES, LANES)] = vals[p]
                st_out[r, pl.ds(PIECES_IN * LANES, LANES)] = tail
                return carry

            lax.fori_loop(0, n_rows, row_body, 0, unroll=False)
            pltpu.sync_copy(st_out.at[pl.ds(0, n_rows)],
                            out_hbm.at[pl.ds(row0, n_rows)])

        def blk_body(j, carry):
            pad_rows(wid * per_w + j * PAD_ROWS, PAD_ROWS)
            return carry

        lax.fori_loop(0, n_blk, blk_body, 0, unroll=False)

        @pl.when(wid == nw - 1)
        def _():
            pad_rows(nw * per_w, n_tail)

    return k(table)


def _emb_kernel(n_rows):
    info = plsc.get_sparse_core_info()
    nc, ns = info.num_cores, info.num_subcores
    nw = nc * ns
    assert n_rows % (nw * CHUNK) == 0
    t_per_w = n_rows // (nw * CHUNK)
    assert t_per_w % NBUF == 0

    mesh = plsc.VectorSubcoreMesh(core_axis_name="c", subcore_axis_name="s")

    @functools.partial(
        pl.kernel,
        mesh=mesh,
        compiler_params=pltpu.CompilerParams(
            use_tc_tiling_on_sc=False, needs_layout_passes=False
        ),
        out_type=jax.ShapeDtypeStruct((n_rows, EMB_PAD), jnp.float32),
        scratch_types=[
            pltpu.VMEM((NBUF, CHUNK), jnp.int32),
            pltpu.VMEM((NBUF, CHUNK, EMB_PAD), jnp.float32),
            pltpu.SemaphoreType.DMA,
            pltpu.SemaphoreType.DMA,
        ],
    )
    def k(tok_hbm, table_hbm, out_hbm, idx_v, rows_v, sem0, sem1):
        sems = (sem0, sem1)
        wid = lax.axis_index("s") * nc + lax.axis_index("c")
        base = wid * t_per_w

        def prime(chunk_id, b):
            pltpu.sync_copy(tok_hbm.at[chunk_id], idx_v.at[b])
            pltpu.async_copy(table_hbm.at[idx_v.at[b]], rows_v.at[b], sems[b])

        def drain(chunk_id, b):
            pltpu.make_async_copy(
                table_hbm.at[idx_v.at[b]], rows_v.at[b], sems[b]
            ).wait()
            pltpu.sync_copy(
                rows_v.at[b], out_hbm.at[pl.ds(chunk_id * CHUNK, CHUNK)]
            )

        for b in range(NBUF):
            prime(base + b, b)

        def body(j, carry):
            for b in range(NBUF):
                i = base + NBUF * j + b
                drain(i, b)
                prime(i + NBUF, b)
            return carry

        lax.fori_loop(0, t_per_w // NBUF - 1, body, 0, unroll=False)

        for b in range(NBUF):
            drain(base + t_per_w - NBUF + b, b)

    return k


def kernel(tokens, table):
    b, l = tokens.shape
    n_rows = b * l
    vocab = table.shape[0]
    tok_flat = tokens.astype(jnp.int32).reshape(n_rows // CHUNK, CHUNK)
    table_pad = _pad_table_sc(table)
    out = _emb_kernel(n_rows)(tok_flat, table_pad)
    return out[:, :EMB].reshape(b, l, EMB)


# speedup vs baseline: 1.1601x; 1.1601x over previous
import functools

import jax
import jax.numpy as jnp
from jax import lax
from jax.experimental import pallas as pl
from jax.experimental.pallas import tpu as pltpu
from jax.experimental.pallas import tpu_sc as plsc

EMB = 300
EMB_PAD = 304
CHUNK = 128
NBUF = 2
PAD_BLOCK = 2048


def _pad_table(table):
    v = table.shape[0]
    grid = (v + PAD_BLOCK - 1) // PAD_BLOCK

    def body(t_ref, o_ref):
        o_ref[...] = jnp.concatenate(
            [t_ref[...], jnp.zeros((PAD_BLOCK, EMB_PAD - EMB), jnp.float32)],
            axis=1,
        )

    return pl.pallas_call(
        body,
        grid=(grid,),
        in_specs=[pl.BlockSpec((PAD_BLOCK, EMB), lambda i: (i, 0))],
        out_specs=pl.BlockSpec((PAD_BLOCK, EMB_PAD), lambda i: (i, 0)),
        out_shape=jax.ShapeDtypeStruct((v, EMB_PAD), jnp.float32),
    )(table)


def _emb_kernel(n_rows):
    info = plsc.get_sparse_core_info()
    nc, ns = info.num_cores, info.num_subcores
    nw = nc * ns
    assert n_rows % (nw * CHUNK) == 0
    t_per_w = n_rows // (nw * CHUNK)
    assert t_per_w % NBUF == 0

    mesh = plsc.VectorSubcoreMesh(core_axis_name="c", subcore_axis_name="s")

    @functools.partial(
        pl.kernel,
        mesh=mesh,
        compiler_params=pltpu.CompilerParams(use_tc_tiling_on_sc=False),
        out_type=jax.ShapeDtypeStruct((n_rows, EMB_PAD), jnp.float32),
        scratch_types=[
            pltpu.VMEM((NBUF, CHUNK), jnp.int32),
            pltpu.VMEM((NBUF, CHUNK, EMB_PAD), jnp.float32),
            pltpu.SemaphoreType.DMA,
            pltpu.SemaphoreType.DMA,
        ],
    )
    def k(tok_hbm, table_hbm, out_hbm, idx_v, rows_v, sem0, sem1):
        sems = (sem0, sem1)
        wid = lax.axis_index("s") * nc + lax.axis_index("c")
        base = wid * t_per_w

        def prime(chunk_id, b):
            pltpu.sync_copy(tok_hbm.at[chunk_id], idx_v.at[b])
            pltpu.async_copy(table_hbm.at[idx_v.at[b]], rows_v.at[b], sems[b])

        def drain(chunk_id, b):
            pltpu.make_async_copy(
                table_hbm.at[idx_v.at[b]], rows_v.at[b], sems[b]
            ).wait()
            pltpu.sync_copy(
                rows_v.at[b], out_hbm.at[pl.ds(chunk_id * CHUNK, CHUNK)]
            )

        for b in range(NBUF):
            prime(base + b, b)

        def body(j, carry):
            for b in range(NBUF):
                i = base + NBUF * j + b
                drain(i, b)
                prime(i + NBUF, b)
            return carry

        lax.fori_loop(0, t_per_w // NBUF - 1, body, 0, unroll=False)

        for b in range(NBUF):
            drain(base + t_per_w - NBUF + b, b)

    return k


def kernel(tokens, table):
    b, l = tokens.shape
    n_rows = b * l
    tok_flat = tokens.astype(jnp.int32).reshape(n_rows // CHUNK, CHUNK)
    table_pad = _pad_table(table)
    out = _emb_kernel(n_rows)(tok_flat, table_pad)
    return out[:, :EMB].reshape(b, l, EMB)


# speedup vs baseline: 1.9862x; 1.7122x over previous
import functools

import jax
import jax.numpy as jnp
from jax import lax
from jax.experimental import pallas as pl
from jax.experimental.pallas import tpu as pltpu
from jax.experimental.pallas import tpu_sc as plsc

EMB = 300
EMB_PAD = 384
CHUNK = 128
NBUF = 2
PAD_BLOCK = 2048


def _pad_table(table):
    v = table.shape[0]
    grid = (v + PAD_BLOCK - 1) // PAD_BLOCK

    def body(t_ref, o_ref):
        o_ref[...] = jnp.concatenate(
            [t_ref[...], jnp.zeros((PAD_BLOCK, EMB_PAD - EMB), jnp.float32)],
            axis=1,
        )

    return pl.pallas_call(
        body,
        grid=(grid,),
        in_specs=[pl.BlockSpec((PAD_BLOCK, EMB), lambda i: (i, 0))],
        out_specs=pl.BlockSpec((PAD_BLOCK, EMB_PAD), lambda i: (i, 0)),
        out_shape=jax.ShapeDtypeStruct((v, EMB_PAD), jnp.float32),
    )(table)


def _emb_kernel(n_rows):
    info = plsc.get_sparse_core_info()
    nc, ns = info.num_cores, info.num_subcores
    nw = nc * ns
    assert n_rows % (nw * CHUNK) == 0
    t_per_w = n_rows // (nw * CHUNK)
    assert t_per_w % NBUF == 0

    mesh = plsc.VectorSubcoreMesh(core_axis_name="c", subcore_axis_name="s")

    @functools.partial(
        pl.kernel,
        mesh=mesh,
        compiler_params=pltpu.CompilerParams(use_tc_tiling_on_sc=True),
        out_type=jax.ShapeDtypeStruct((n_rows, EMB_PAD), jnp.float32),
        scratch_types=[
            pltpu.VMEM((NBUF, CHUNK), jnp.int32),
            pltpu.VMEM((NBUF, CHUNK, EMB_PAD), jnp.float32),
            pltpu.SemaphoreType.DMA,
            pltpu.SemaphoreType.DMA,
        ],
    )
    def k(tok_hbm, table_hbm, out_hbm, idx_v, rows_v, sem0, sem1):
        sems = (sem0, sem1)
        wid = lax.axis_index("s") * nc + lax.axis_index("c")
        base = wid * t_per_w

        def prime(chunk_id, b):
            pltpu.sync_copy(tok_hbm.at[chunk_id], idx_v.at[b])
            pltpu.async_copy(table_hbm.at[idx_v.at[b]], rows_v.at[b], sems[b])

        def drain(chunk_id, b):
            pltpu.make_async_copy(
                table_hbm.at[idx_v.at[b]], rows_v.at[b], sems[b]
            ).wait()
            pltpu.sync_copy(
                rows_v.at[b], out_hbm.at[pl.ds(chunk_id * CHUNK, CHUNK)]
            )

        for b in range(NBUF):
            prime(base + b, b)

        def body(j, carry):
            for b in range(NBUF):
                i = base + NBUF * j + b
                drain(i, b)
                prime(i + NBUF, b)
            return carry

        lax.fori_loop(0, t_per_w // NBUF - 1, body, 0, unroll=False)

        for b in range(NBUF):
            drain(base + t_per_w - NBUF + b, b)

    return k


def kernel(tokens, table):
    b, l = tokens.shape
    n_rows = b * l
    tok_flat = tokens.astype(jnp.int32).reshape(n_rows // CHUNK, CHUNK)
    table_pad = _pad_table(table)
    out = _emb_kernel(n_rows)(tok_flat, table_pad)
    return out[:, :EMB].reshape(b, l, EMB)
